# baseline (speedup 1.0000x reference)
import functools

import jax
import jax.numpy as jnp
from jax import lax
from jax.experimental import pallas as pl
from jax.experimental.pallas import tpu as pltpu
from jax.experimental.pallas import tpu_sc as plsc

NC = 2
NS = 16
L = 16
NW = NC * NS

TC_ROWS = 2560
TC_BLK = 512


def _sc_make_body(rows_per_w, d, b, row_off):
    nblk = rows_per_w // b
    nch = d // L

    def compute(perm_v, in_v, out_v):
        @plsc.parallel_loop(0, nch, unroll=4)
        def _chunk(i):
            idx = perm_v[pl.ds(i * L, L)]
            for r in range(b):
                row = jnp.full((L,), r, dtype=jnp.int32)
                out_v[r, pl.ds(i * L, L)] = plsc.load_gather(
                    in_v, [row, idx])

    def body(x_hbm, perm_hbm, out_hbm, perm_v,
             in0, in1, out0, out1, sin0, sin1, sout0, sout1):
        c = lax.axis_index("c")
        s = lax.axis_index("s")
        wid = s * NC + c
        base = row_off + wid * rows_per_w
        pltpu.sync_copy(perm_hbm, perm_v)
        ins, outs = (in0, in1), (out0, out1)
        sins, souts = (sin0, sin1), (sout0, sout1)

        def in_copy(blk, j):
            return pltpu.make_async_copy(
                x_hbm.at[pl.ds(base + blk * b, b), :],
                ins[j], sins[j])

        def out_copy(blk, j):
            return pltpu.make_async_copy(
                outs[j],
                out_hbm.at[pl.ds((base - row_off) + blk * b, b), :],
                souts[j])

        in_copy(0, 0).start()
        in_copy(1, 1).start()

        def pair(t, carry):
            for j in range(2):
                blk = t * 2 + j
                in_copy(blk, j).wait()

                @pl.when(blk >= 2)
                def _wait_out():
                    out_copy(blk - 2, j).wait()

                compute(perm_v, ins[j], outs[j])
                out_copy(blk, j).start()

                @pl.when(blk + 2 < nblk)
                def _prefetch():
                    in_copy(blk + 2, j).start()
            return carry

        lax.fori_loop(0, nblk // 2, pair, 0, unroll=False)
        out_copy(nblk - 2, 0).wait()
        out_copy(nblk - 1, 1).wait()

    return body


def _sc_permute(x2, perm, rows, d, b, row_off, sc_rows):
    rows_per_w = sc_rows // NW
    body = _sc_make_body(rows_per_w, d, b, row_off)
    return pl.kernel(
        body,
        out_type=jax.ShapeDtypeStruct((sc_rows, d), jnp.float32),
        mesh=plsc.VectorSubcoreMesh(
            core_axis_name="c", subcore_axis_name="s",
            num_cores=NC, num_subcores=NS),
        scratch_types=[
            pltpu.VMEM((d,), jnp.int32),
            pltpu.VMEM((b, d), jnp.float32),
            pltpu.VMEM((b, d), jnp.float32),
            pltpu.VMEM((b, d), jnp.float32),
            pltpu.VMEM((b, d), jnp.float32),
            pltpu.SemaphoreType.DMA,
            pltpu.SemaphoreType.DMA,
            pltpu.SemaphoreType.DMA,
            pltpu.SemaphoreType.DMA,
        ],
        compiler_params=pltpu.CompilerParams(needs_layout_passes=False),
    )(x2, perm)


def _tc_body(d, perm_ref, x_ref, o_ref):
    x = x_ref[...]
    ncg = d // 128
    for cg in range(ncg):
        idx = perm_ref[:, cg * 128:(cg + 1) * 128]
        terms = []
        for j in range(ncg):
            local = jnp.clip(idx - j * 128, 0, 127)
            g = jnp.take_along_axis(
                x[:, j * 128:(j + 1) * 128],
                jnp.broadcast_to(local, (x.shape[0], 128)), axis=1)
            mask = jnp.broadcast_to(
                (idx >= j * 128) & (idx < (j + 1) * 128),
                (x.shape[0], 128))
            terms.append(jnp.where(mask, g, jnp.float32(0.0)))
        while len(terms) > 1:
            terms = [a + b for a, b in zip(terms[::2], terms[1::2])]
        o_ref[:, cg * 128:(cg + 1) * 128] = terms[0]


def _tc_permute(x2, perm2, d):
    body = functools.partial(_tc_body, d)
    return pl.pallas_call(
        body,
        out_shape=jax.ShapeDtypeStruct((TC_ROWS, d), jnp.float32),
        grid=(TC_ROWS // TC_BLK,),
        in_specs=[
            pl.BlockSpec((1, d), lambda i: (0, 0)),
            pl.BlockSpec((TC_BLK, d), lambda i: (i, 0)),
        ],
        out_specs=pl.BlockSpec((TC_BLK, d), lambda i: (i, 0)),
    )(perm2, x2[:TC_ROWS])


def kernel(x, permutation):
    lead = x.shape[:-1]
    d = x.shape[-1]
    rows = 1
    for n in lead:
        rows *= n
    x2 = x.reshape(rows, d)
    perm = permutation.astype(jnp.int32)
    y_sc = _sc_permute(x2, perm, rows, d, 8, TC_ROWS, rows - TC_ROWS)
    y_tc = _tc_permute(x2, perm.reshape(1, d), d)
    out = jnp.concatenate([y_tc, y_sc], axis=0)
    return out.reshape(x.shape)


# speedup vs baseline: 1.9087x; 1.9087x over previous
import jax
import jax.numpy as jnp
from jax import lax
from jax.experimental import pallas as pl
from jax.experimental.pallas import tpu as pltpu
from jax.experimental.pallas import tpu_sc as plsc

NC = 2
NS = 16
L = 16
NW = NC * NS


def _make_body(rows_per_w, d, b):
    nblk = rows_per_w // b
    nch = d // L
    blk_elems = b * d

    def compute(perm_v, in_v, out_v):
        @plsc.parallel_loop(0, nch, unroll=4)
        def _chunk(i):
            idx = perm_v[pl.ds(i * L, L)]
            for r in range(b):
                row = jnp.full((L,), r, dtype=jnp.int32)
                out_v[r, pl.ds(i * L, L)] = plsc.load_gather(
                    in_v, [row, idx])

    def body(x_hbm, perm_hbm, out_hbm, perm_v,
             in0, in1, out0, out1, sin0, sin1, sout0, sout1):
        c = lax.axis_index("c")
        s = lax.axis_index("s")
        wid = s * NC + c
        base = wid * rows_per_w
        pltpu.sync_copy(perm_hbm, perm_v)
        ins, outs = (in0, in1), (out0, out1)
        sins, souts = (sin0, sin1), (sout0, sout1)

        def in_copy(blk, j):
            return pltpu.make_async_copy(
                x_hbm.at[pl.ds(base + blk * b, b), :],
                ins[j], sins[j])

        def out_copy(blk, j):
            return pltpu.make_async_copy(
                outs[j],
                out_hbm.at[pl.ds(base + blk * b, b), :],
                souts[j])

        in_copy(0, 0).start()
        in_copy(1, 1).start()

        def pair(t, carry):
            for j in range(2):
                blk = t * 2 + j
                in_copy(blk, j).wait()

                @pl.when(blk >= 2)
                def _wait_out():
                    out_copy(blk - 2, j).wait()

                compute(perm_v, ins[j], outs[j])
                out_copy(blk, j).start()

                @pl.when(blk + 2 < nblk)
                def _prefetch():
                    in_copy(blk + 2, j).start()
            return carry

        lax.fori_loop(0, nblk // 2, pair, 0, unroll=False)
        out_copy(nblk - 2, 0).wait()
        out_copy(nblk - 1, 1).wait()

    return body


def _permute(x2, perm, rows, d, b):
    rows_per_w = rows // NW
    body = _make_body(rows_per_w, d, b)
    return pl.kernel(
        body,
        out_type=jax.ShapeDtypeStruct((rows, d), jnp.float32),
        mesh=plsc.VectorSubcoreMesh(
            core_axis_name="c", subcore_axis_name="s",
            num_cores=NC, num_subcores=NS),
        scratch_types=[
            pltpu.VMEM((d,), jnp.int32),
            pltpu.VMEM((b, d), jnp.float32),
            pltpu.VMEM((b, d), jnp.float32),
            pltpu.VMEM((b, d), jnp.float32),
            pltpu.VMEM((b, d), jnp.float32),
            pltpu.SemaphoreType.DMA,
            pltpu.SemaphoreType.DMA,
            pltpu.SemaphoreType.DMA,
            pltpu.SemaphoreType.DMA,
        ],
        compiler_params=pltpu.CompilerParams(needs_layout_passes=False),
    )(x2, perm)


def kernel(x, permutation):
    lead = x.shape[:-1]
    d = x.shape[-1]
    rows = 1
    for n in lead:
        rows *= n
    x2 = x.reshape(rows, d)
    perm = permutation.astype(jnp.int32)
    out = _permute(x2, perm, rows, d, b=8)
    return out.reshape(x.shape)
